# Initial kernel scaffold; baseline (speedup 1.0000x reference)
import jax
import jax.numpy as jnp
from jax.experimental import pallas as pl


def kernel(h, pos, edge_index, emb_w, emb_b, mlp1_w, mlp1_b, mlp2_w, mlp2_b, lin1_w, lin2_w, lin2_b, lin_w, lin_b, out1_w, out1_b, out2_w, out2_b):
    raise NotImplementedError("write your pallas kernel here")


# speedup vs baseline: 2.4623x; 2.4623x over previous
import functools
import math

import jax
import jax.numpy as jnp
from jax import lax
from jax.experimental import pallas as pl
from jax.experimental.pallas import tpu as pltpu
from jax.experimental.pallas import tpu_sc as plsc

N = 10000
E = 320000
D = 128
NG = 50
GP = 64
L = 6
CUTOFF = 10.0
PI = math.pi
COEFF = -0.5 / (CUTOFF / (NG - 1)) ** 2
LOG2 = math.log(2.0)

NC = 2
NS = 16
NW = NC * NS
EPT = E // NW
CH = 80
NCHUNK = EPT // CH
NP = 10240
RPT = NP // NS

NB = 1000
BE = 3200

_mesh = plsc.VectorSubcoreMesh(core_axis_name="c", subcore_axis_name="s")


def _ssp(x):
    return jnp.maximum(x, 0.0) + jnp.log1p(jnp.exp(-jnp.abs(x))) - LOG2


@functools.partial(
    pl.kernel,
    mesh=_mesh,
    compiler_params=pltpu.CompilerParams(needs_layout_passes=False),
    out_type=jax.ShapeDtypeStruct((E,), jnp.float32),
    scratch_types=[
        pltpu.VMEM((3 * N,), jnp.float32),
        pltpu.VMEM((EPT,), jnp.int32),
        pltpu.VMEM((EPT,), jnp.int32),
        pltpu.VMEM((EPT,), jnp.float32),
    ],
)
def _sq_dist_sc(posf_hbm, row_hbm, col_hbm, sq_hbm, pos_v, row_v, col_v, out_v):
    cid = lax.axis_index("c")
    sid = lax.axis_index("s")
    wid = sid * NC + cid
    base = wid * EPT
    pltpu.sync_copy(posf_hbm, pos_v)
    pltpu.sync_copy(row_hbm.at[pl.ds(base, EPT)], row_v)
    pltpu.sync_copy(col_hbm.at[pl.ds(base, EPT)], col_v)

    def body(j, carry):
        r3 = row_v[pl.ds(j * 16, 16)] * 3
        c3 = col_v[pl.ds(j * 16, 16)] * 3
        dx = plsc.load_gather(pos_v, [r3]) - plsc.load_gather(pos_v, [c3])
        dy = plsc.load_gather(pos_v, [r3 + 1]) - plsc.load_gather(pos_v, [c3 + 1])
        dz = plsc.load_gather(pos_v, [r3 + 2]) - plsc.load_gather(pos_v, [c3 + 2])
        out_v[pl.ds(j * 16, 16)] = dx * dx + dy * dy + dz * dz
        return carry

    lax.fori_loop(0, EPT // 16, body, 0)
    pltpu.sync_copy(out_v, sq_hbm.at[pl.ds(base, EPT)])


@functools.partial(
    pl.kernel,
    mesh=_mesh,
    compiler_params=pltpu.CompilerParams(needs_layout_passes=False),
    out_type=jax.ShapeDtypeStruct((NC, NP, D), jnp.float32),
    scratch_types=[
        pltpu.VMEM((CH,), jnp.int32),
        pltpu.VMEM((CH,), jnp.int32),
        pltpu.VMEM((CH, D), jnp.float32),
        pltpu.VMEM((CH, D), jnp.float32),
        pltpu.VMEM_SHARED((NP, D), jnp.float32),
        pltpu.SemaphoreType.DMA,
    ],
)
def _gather_mul_scatter_sc(x_hbm, filt_hbm, row_hbm, col_hbm, zeros_hbm,
                           out_hbm, row_v, col_v, xg_v, f_v, agg_sh, sem):
    cid = lax.axis_index("c")
    sid = lax.axis_index("s")
    wid = sid * NC + cid
    pltpu.sync_copy(zeros_hbm.at[pl.ds(sid * RPT, RPT)],
                    agg_sh.at[pl.ds(sid * RPT, RPT)])
    plsc.subcore_barrier()

    def chunk(ci, carry):
        base = wid * EPT + ci * CH
        pltpu.sync_copy(row_hbm.at[pl.ds(base, CH)], row_v)
        pltpu.sync_copy(col_hbm.at[pl.ds(base, CH)], col_v)
        pltpu.async_copy(x_hbm.at[row_v], xg_v, sem).wait()
        pltpu.sync_copy(filt_hbm.at[pl.ds(base, CH)], f_v)

        def mrow(e, c2):
            for k in range(D // 16):
                sl = pl.ds(k * 16, 16)
                xg_v[e, sl] = xg_v[e, sl] * f_v[e, sl]
            return c2

        lax.fori_loop(0, CH, mrow, 0)
        pltpu.sync_copy(xg_v, agg_sh.at[col_v], add=True)
        return carry

    lax.fori_loop(0, NCHUNK, chunk, 0)
    plsc.subcore_barrier()
    pltpu.sync_copy(agg_sh.at[pl.ds(sid * RPT, RPT)],
                    out_hbm.at[cid, pl.ds(sid * RPT, RPT)])


def _embed_body(h_ref, w_ref, b_ref, o_ref):
    o_ref[...] = (jnp.dot(h_ref[...], w_ref[...],
                          preferred_element_type=jnp.float32) + b_ref[...])


def _embed_tc(h, w, b):
    return pl.pallas_call(
        _embed_body,
        grid=(N // NB,),
        in_specs=[
            pl.BlockSpec((NB, D), lambda i: (i, 0)),
            pl.BlockSpec((D, D), lambda i: (0, 0)),
            pl.BlockSpec((1, D), lambda i: (0, 0)),
        ],
        out_specs=pl.BlockSpec((NB, D), lambda i: (i, 0)),
        out_shape=jax.ShapeDtypeStruct((N, D), jnp.float32),
    )(h, w, b)


def _ea_body(sq_ref, ea_ref, c_ref):
    ew = jnp.sqrt(sq_ref[...] + 1e-12)
    off = (lax.broadcasted_iota(jnp.int32, (1, GP), 1).astype(jnp.float32)
           * (CUTOFF / (NG - 1)))
    ea_ref[...] = jnp.exp(COEFF * (ew - off) ** 2)
    c_ref[...] = 0.5 * (jnp.cos(ew * (PI / CUTOFF)) + 1.0)


def _ea_tc(sq):
    return pl.pallas_call(
        _ea_body,
        grid=(E // BE,),
        in_specs=[pl.BlockSpec((BE, 1), lambda i: (i, 0))],
        out_specs=[
            pl.BlockSpec((BE, GP), lambda i: (i, 0)),
            pl.BlockSpec((BE, 1), lambda i: (i, 0)),
        ],
        out_shape=[
            jax.ShapeDtypeStruct((E, GP), jnp.float32),
            jax.ShapeDtypeStruct((E, 1), jnp.float32),
        ],
    )(sq)


def _filt_body(ea_ref, c_ref, w1_ref, b1_ref, w2_ref, b2_ref, o_ref):
    t = jnp.dot(ea_ref[...], w1_ref[...],
                preferred_element_type=jnp.float32) + b1_ref[...]
    t = _ssp(t)
    f = jnp.dot(t, w2_ref[...],
                preferred_element_type=jnp.float32) + b2_ref[...]
    o_ref[...] = f * c_ref[...]


def _filt_tc(ea, c, w1, b1, w2, b2):
    return pl.pallas_call(
        _filt_body,
        grid=(E // BE,),
        in_specs=[
            pl.BlockSpec((BE, GP), lambda i: (i, 0)),
            pl.BlockSpec((BE, 1), lambda i: (i, 0)),
            pl.BlockSpec((GP, D), lambda i: (0, 0)),
            pl.BlockSpec((1, D), lambda i: (0, 0)),
            pl.BlockSpec((D, D), lambda i: (0, 0)),
            pl.BlockSpec((1, D), lambda i: (0, 0)),
        ],
        out_specs=pl.BlockSpec((BE, D), lambda i: (i, 0)),
        out_shape=jax.ShapeDtypeStruct((E, D), jnp.float32),
    )(ea, c, w1, b1, w2, b2)


def _mm_body(a_ref, w_ref, o_ref):
    o_ref[...] = jnp.dot(a_ref[...], w_ref[...],
                         preferred_element_type=jnp.float32)


def _mm_tc(a, w):
    return pl.pallas_call(
        _mm_body,
        grid=(N // NB,),
        in_specs=[
            pl.BlockSpec((NB, D), lambda i: (i, 0)),
            pl.BlockSpec((D, D), lambda i: (0, 0)),
        ],
        out_specs=pl.BlockSpec((NB, D), lambda i: (i, 0)),
        out_shape=jax.ShapeDtypeStruct((N, D), jnp.float32),
    )(a, w)


def _update_body(a0_ref, a1_ref, h_ref, w2_ref, b2_ref, ww_ref, wb_ref, o_ref):
    agg = a0_ref[...] + a1_ref[...]
    u = jnp.dot(agg, w2_ref[...],
                preferred_element_type=jnp.float32) + b2_ref[...]
    u = _ssp(u)
    u = jnp.dot(u, ww_ref[...],
                preferred_element_type=jnp.float32) + wb_ref[...]
    o_ref[...] = h_ref[...] + u


def _update_tc(a0, a1, h, w2, b2, ww, wb):
    return pl.pallas_call(
        _update_body,
        grid=(N // NB,),
        in_specs=[
            pl.BlockSpec((NB, D), lambda i: (i, 0)),
            pl.BlockSpec((NB, D), lambda i: (i, 0)),
            pl.BlockSpec((NB, D), lambda i: (i, 0)),
            pl.BlockSpec((D, D), lambda i: (0, 0)),
            pl.BlockSpec((1, D), lambda i: (0, 0)),
            pl.BlockSpec((D, D), lambda i: (0, 0)),
            pl.BlockSpec((1, D), lambda i: (0, 0)),
        ],
        out_specs=pl.BlockSpec((NB, D), lambda i: (i, 0)),
        out_shape=jax.ShapeDtypeStruct((N, D), jnp.float32),
    )(a0, a1, h, w2, b2, ww, wb)


def _head_body(h_ref, w1_ref, b1_ref, w2_ref, b2_ref, o_ref):
    t = jnp.dot(h_ref[...], w1_ref[...],
                preferred_element_type=jnp.float32) + b1_ref[...]
    t = _ssp(t)
    o_ref[...] = jnp.dot(t, w2_ref[...],
                         preferred_element_type=jnp.float32) + b2_ref[...]


def _head_tc(h, w1, b1, w2, b2):
    return pl.pallas_call(
        _head_body,
        grid=(N // NB,),
        in_specs=[
            pl.BlockSpec((NB, D), lambda i: (i, 0)),
            pl.BlockSpec((D, D // 2), lambda i: (0, 0)),
            pl.BlockSpec((1, D // 2), lambda i: (0, 0)),
            pl.BlockSpec((D // 2, D), lambda i: (0, 0)),
            pl.BlockSpec((1, D), lambda i: (0, 0)),
        ],
        out_specs=pl.BlockSpec((NB, D), lambda i: (i, 0)),
        out_shape=jax.ShapeDtypeStruct((N, D), jnp.float32),
    )(h, w1, b1, w2, b2)


def kernel(h, pos, edge_index, emb_w, emb_b, mlp1_w, mlp1_b, mlp2_w, mlp2_b,
           lin1_w, lin2_w, lin2_b, lin_w, lin_b, out1_w, out1_b, out2_w,
           out2_b):
    row = edge_index[0]
    col = edge_index[1]
    posf = pos.reshape(-1)
    zeros = jnp.zeros((NP, D), jnp.float32)

    sq = _sq_dist_sc(posf, row, col)
    ea, cg = _ea_tc(sq.reshape(E, 1))
    hcur = _embed_tc(h, emb_w, emb_b.reshape(1, D))

    w1p = jnp.concatenate(
        [mlp1_w, jnp.zeros((L, GP - NG, D), jnp.float32)], axis=1)

    for i in range(L):
        filt = _filt_tc(ea, cg, w1p[i], mlp1_b[i].reshape(1, D),
                        mlp2_w[i], mlp2_b[i].reshape(1, D))
        x = _mm_tc(hcur, lin1_w[i])
        parts = _gather_mul_scatter_sc(x, filt, row, col, zeros)
        hcur = _update_tc(parts[0, :N], parts[1, :N], hcur, lin2_w[i],
                          lin2_b[i].reshape(1, D), lin_w[i],
                          lin_b[i].reshape(1, D))

    return _head_tc(hcur, out1_w, out1_b.reshape(1, D // 2),
                    out2_w, out2_b.reshape(1, D))


# speedup vs baseline: 2.7533x; 1.1182x over previous
import functools
import math

import jax
import jax.numpy as jnp
import numpy as np
from jax import lax
from jax.experimental import pallas as pl
from jax.experimental.pallas import tpu as pltpu
from jax.experimental.pallas import tpu_sc as plsc

N = 10000
E = 320000
D = 128
NG = 50
GP = 64
L = 6
CUTOFF = 10.0
PI = math.pi
COEFF = -0.5 / (CUTOFF / (NG - 1)) ** 2
LOG2 = math.log(2.0)

NC = 2
NS = 16
NW = NC * NS
EPT = E // NW
CH = 40
NCHUNK = EPT // CH
SLAB = 640
SLABL = N - 15 * SLAB

NB = 1000
BE = 3200

_Q = np.zeros((D,), dtype=np.int32)
for _g in range(D // 32):
    for _j in range(16):
        _Q[32 * _g + 2 * _j] = 32 * _g + _j
        _Q[32 * _g + 2 * _j + 1] = 32 * _g + 16 + _j

_mesh = plsc.VectorSubcoreMesh(core_axis_name="c", subcore_axis_name="s")


def _ssp(x):
    return jnp.maximum(x, 0.0) + jnp.log1p(jnp.exp(-jnp.abs(x))) - LOG2


@functools.partial(
    pl.kernel,
    mesh=_mesh,
    compiler_params=pltpu.CompilerParams(needs_layout_passes=False),
    out_type=jax.ShapeDtypeStruct((E,), jnp.float32),
    scratch_types=[
        pltpu.VMEM((3 * N,), jnp.float32),
        pltpu.VMEM((EPT,), jnp.int32),
        pltpu.VMEM((EPT,), jnp.int32),
        pltpu.VMEM((EPT,), jnp.float32),
    ],
)
def _sq_dist_sc(posf_hbm, row_hbm, col_hbm, sq_hbm, pos_v, row_v, col_v, out_v):
    cid = lax.axis_index("c")
    sid = lax.axis_index("s")
    wid = sid * NC + cid
    base = wid * EPT
    pltpu.sync_copy(posf_hbm, pos_v)
    pltpu.sync_copy(row_hbm.at[pl.ds(base, EPT)], row_v)
    pltpu.sync_copy(col_hbm.at[pl.ds(base, EPT)], col_v)

    def body(j, carry):
        r3 = row_v[pl.ds(j * 16, 16)] * 3
        c3 = col_v[pl.ds(j * 16, 16)] * 3
        dx = plsc.load_gather(pos_v, [r3]) - plsc.load_gather(pos_v, [c3])
        dy = plsc.load_gather(pos_v, [r3 + 1]) - plsc.load_gather(pos_v, [c3 + 1])
        dz = plsc.load_gather(pos_v, [r3 + 2]) - plsc.load_gather(pos_v, [c3 + 2])
        out_v[pl.ds(j * 16, 16)] = dx * dx + dy * dy + dz * dz
        return carry

    lax.fori_loop(0, EPT // 16, body, 0)
    pltpu.sync_copy(out_v, sq_hbm.at[pl.ds(base, EPT)])


@functools.partial(
    pl.kernel,
    mesh=_mesh,
    compiler_params=pltpu.CompilerParams(needs_layout_passes=False),
    out_type=jax.ShapeDtypeStruct((NC, N, D), jnp.float32),
    scratch_types=[
        pltpu.VMEM((EPT,), jnp.int32),
        pltpu.VMEM((EPT,), jnp.int32),
        pltpu.VMEM((CH, D), jnp.float32),
        pltpu.VMEM((CH, D), jnp.float32),
        pltpu.VMEM((CH, D), jnp.float32),
        pltpu.VMEM((CH, D), jnp.float32),
        pltpu.VMEM((CH, D), jnp.float32),
        pltpu.VMEM_SHARED((N, D), jnp.float32),
        pltpu.SemaphoreType.DMA,
        pltpu.SemaphoreType.DMA,
        pltpu.SemaphoreType.DMA,
        pltpu.SemaphoreType.DMA,
        pltpu.SemaphoreType.DMA,
    ],
)
def _gather_mul_scatter_sc(x_hbm, filt_hbm, row_hbm, col_hbm, zeros_hbm,
                           out_hbm, row_v, col_v, xg0, xg1, f0, f1, msg,
                           agg_sh, g0, g1, q0, q1, ssem):
    cid = lax.axis_index("c")
    sid = lax.axis_index("s")
    wid = sid * NC + cid
    ebase = wid * EPT
    xgs, fs = (xg0, xg1), (f0, f1)
    gsem, qsem = (g0, g1), (q0, q1)

    @pl.when(sid < NS - 1)
    def _():
        pltpu.sync_copy(zeros_hbm.at[pl.ds(sid * SLAB, SLAB)],
                        agg_sh.at[pl.ds(sid * SLAB, SLAB)])

    @pl.when(sid == NS - 1)
    def _():
        pltpu.sync_copy(zeros_hbm.at[pl.ds((NS - 1) * SLAB, SLABL)],
                        agg_sh.at[pl.ds((NS - 1) * SLAB, SLABL)])

    pltpu.sync_copy(row_hbm.at[pl.ds(ebase, EPT)], row_v)
    pltpu.sync_copy(col_hbm.at[pl.ds(ebase, EPT)], col_v)
    plsc.subcore_barrier()

    def start_in(c, p):
        pltpu.async_copy(x_hbm.at[row_v.at[pl.ds(c * CH, CH)]], xgs[p],
                         gsem[p])
        pltpu.async_copy(filt_hbm.at[pl.ds(ebase + c * CH, CH)], fs[p],
                         qsem[p])

    def wait_in(p):
        pltpu.make_async_copy(x_hbm.at[row_v.at[pl.ds(0, CH)]], xgs[p],
                              gsem[p]).wait()
        pltpu.make_async_copy(filt_hbm.at[pl.ds(0, CH)], fs[p],
                              qsem[p]).wait()

    def start_sc(c):
        pltpu.async_copy(msg, agg_sh.at[col_v.at[pl.ds(c * CH, CH)]], ssem,
                         add=True)

    def wait_sc():
        pltpu.make_async_copy(msg, agg_sh.at[col_v.at[pl.ds(0, CH)]],
                              ssem).wait()

    def mul(p):
        a, b = xgs[p], fs[p]

        def mrow(e, c2):
            for g in range(D // 16):
                sl = pl.ds(16 * g, 16)
                msg[e, sl] = a[e, sl] * b[e, sl]
            return c2

        lax.fori_loop(0, CH, mrow, 0, unroll=2)

    start_in(0, 0)
    start_in(1, 1)

    def pair(ci, carry):
        c0 = 2 * ci

        wait_in(0)

        @pl.when(ci > 0)
        def _():
            wait_sc()

        mul(0)

        @pl.when(ci < NCHUNK // 2 - 1)
        def _():
            start_in(c0 + 2, 0)

        start_sc(c0)

        wait_in(1)
        wait_sc()
        mul(1)

        @pl.when(ci < NCHUNK // 2 - 1)
        def _():
            start_in(c0 + 3, 1)

        start_sc(c0 + 1)
        return carry

    lax.fori_loop(0, NCHUNK // 2, pair, 0)
    wait_sc()

    plsc.subcore_barrier()

    @pl.when(sid < NS - 1)
    def _():
        pltpu.sync_copy(agg_sh.at[pl.ds(sid * SLAB, SLAB)],
                        out_hbm.at[cid, pl.ds(sid * SLAB, SLAB)])

    @pl.when(sid == NS - 1)
    def _():
        pltpu.sync_copy(agg_sh.at[pl.ds((NS - 1) * SLAB, SLABL)],
                        out_hbm.at[cid, pl.ds((NS - 1) * SLAB, SLABL)])


def _embed_body(h_ref, w_ref, b_ref, o_ref):
    o_ref[...] = (jnp.dot(h_ref[...], w_ref[...],
                          preferred_element_type=jnp.float32) + b_ref[...])


def _embed_tc(h, w, b):
    return pl.pallas_call(
        _embed_body,
        grid=(N // NB,),
        in_specs=[
            pl.BlockSpec((NB, D), lambda i: (i, 0)),
            pl.BlockSpec((D, D), lambda i: (0, 0)),
            pl.BlockSpec((1, D), lambda i: (0, 0)),
        ],
        out_specs=pl.BlockSpec((NB, D), lambda i: (i, 0)),
        out_shape=jax.ShapeDtypeStruct((N, D), jnp.float32),
    )(h, w, b)


def _ea_body(sq_ref, ea_ref, c_ref):
    ew = jnp.sqrt(sq_ref[...] + 1e-12)
    off = (lax.broadcasted_iota(jnp.int32, (1, GP), 1).astype(jnp.float32)
           * (CUTOFF / (NG - 1)))
    ea_ref[...] = jnp.exp(COEFF * (ew - off) ** 2)
    c_ref[...] = 0.5 * (jnp.cos(ew * (PI / CUTOFF)) + 1.0)


def _ea_tc(sq):
    return pl.pallas_call(
        _ea_body,
        grid=(E // BE,),
        in_specs=[pl.BlockSpec((BE, 1), lambda i: (i, 0))],
        out_specs=[
            pl.BlockSpec((BE, GP), lambda i: (i, 0)),
            pl.BlockSpec((BE, 1), lambda i: (i, 0)),
        ],
        out_shape=[
            jax.ShapeDtypeStruct((E, GP), jnp.float32),
            jax.ShapeDtypeStruct((E, 1), jnp.float32),
        ],
    )(sq)


def _filt_body(ea_ref, c_ref, w1_ref, b1_ref, w2_ref, b2_ref, o_ref):
    t = jnp.dot(ea_ref[...], w1_ref[...],
                preferred_element_type=jnp.float32) + b1_ref[...]
    t = _ssp(t)
    f = jnp.dot(t, w2_ref[...],
                preferred_element_type=jnp.float32) + b2_ref[...]
    o_ref[...] = f * c_ref[...]


def _filt_tc(ea, c, w1, b1, w2, b2):
    return pl.pallas_call(
        _filt_body,
        grid=(E // BE,),
        in_specs=[
            pl.BlockSpec((BE, GP), lambda i: (i, 0)),
            pl.BlockSpec((BE, 1), lambda i: (i, 0)),
            pl.BlockSpec((GP, D), lambda i: (0, 0)),
            pl.BlockSpec((1, D), lambda i: (0, 0)),
            pl.BlockSpec((D, D), lambda i: (0, 0)),
            pl.BlockSpec((1, D), lambda i: (0, 0)),
        ],
        out_specs=pl.BlockSpec((BE, D), lambda i: (i, 0)),
        out_shape=jax.ShapeDtypeStruct((E, D), jnp.float32),
    )(ea, c, w1, b1, w2, b2)


def _mm_body(a_ref, w_ref, o_ref):
    o_ref[...] = jnp.dot(a_ref[...], w_ref[...],
                         preferred_element_type=jnp.float32)


def _mm_tc(a, w):
    return pl.pallas_call(
        _mm_body,
        grid=(N // NB,),
        in_specs=[
            pl.BlockSpec((NB, D), lambda i: (i, 0)),
            pl.BlockSpec((D, D), lambda i: (0, 0)),
        ],
        out_specs=pl.BlockSpec((NB, D), lambda i: (i, 0)),
        out_shape=jax.ShapeDtypeStruct((N, D), jnp.float32),
    )(a, w)


def _update_body(a0_ref, a1_ref, h_ref, w2_ref, b2_ref, ww_ref, wb_ref, o_ref):
    agg = a0_ref[...] + a1_ref[...]
    u = jnp.dot(agg, w2_ref[...],
                preferred_element_type=jnp.float32) + b2_ref[...]
    u = _ssp(u)
    u = jnp.dot(u, ww_ref[...],
                preferred_element_type=jnp.float32) + wb_ref[...]
    o_ref[...] = h_ref[...] + u


def _update_tc(a0, a1, h, w2, b2, ww, wb):
    return pl.pallas_call(
        _update_body,
        grid=(N // NB,),
        in_specs=[
            pl.BlockSpec((NB, D), lambda i: (i, 0)),
            pl.BlockSpec((NB, D), lambda i: (i, 0)),
            pl.BlockSpec((NB, D), lambda i: (i, 0)),
            pl.BlockSpec((D, D), lambda i: (0, 0)),
            pl.BlockSpec((1, D), lambda i: (0, 0)),
            pl.BlockSpec((D, D), lambda i: (0, 0)),
            pl.BlockSpec((1, D), lambda i: (0, 0)),
        ],
        out_specs=pl.BlockSpec((NB, D), lambda i: (i, 0)),
        out_shape=jax.ShapeDtypeStruct((N, D), jnp.float32),
    )(a0, a1, h, w2, b2, ww, wb)


def _head_body(h_ref, w1_ref, b1_ref, w2_ref, b2_ref, o_ref):
    t = jnp.dot(h_ref[...], w1_ref[...],
                preferred_element_type=jnp.float32) + b1_ref[...]
    t = _ssp(t)
    o_ref[...] = jnp.dot(t, w2_ref[...],
                         preferred_element_type=jnp.float32) + b2_ref[...]


def _head_tc(h, w1, b1, w2, b2):
    return pl.pallas_call(
        _head_body,
        grid=(N // NB,),
        in_specs=[
            pl.BlockSpec((NB, D), lambda i: (i, 0)),
            pl.BlockSpec((D, D // 2), lambda i: (0, 0)),
            pl.BlockSpec((1, D // 2), lambda i: (0, 0)),
            pl.BlockSpec((D // 2, D), lambda i: (0, 0)),
            pl.BlockSpec((1, D), lambda i: (0, 0)),
        ],
        out_specs=pl.BlockSpec((NB, D), lambda i: (i, 0)),
        out_shape=jax.ShapeDtypeStruct((N, D), jnp.float32),
    )(h, w1, b1, w2, b2)


def kernel(h, pos, edge_index, emb_w, emb_b, mlp1_w, mlp1_b, mlp2_w, mlp2_b,
           lin1_w, lin2_w, lin2_b, lin_w, lin_b, out1_w, out1_b, out2_w,
           out2_b):
    row = edge_index[0]
    col = edge_index[1]
    posf = pos.reshape(-1)
    zeros = jnp.zeros((N, D), jnp.float32)

    sq = _sq_dist_sc(posf, row, col)
    ea, cg = _ea_tc(sq.reshape(E, 1))
    hcur = _embed_tc(h, emb_w, emb_b.reshape(1, D))

    w1p = jnp.concatenate(
        [mlp1_w, jnp.zeros((L, GP - NG, D), jnp.float32)], axis=1)

    filts = [_filt_tc(ea, cg, w1p[i], mlp1_b[i].reshape(1, D),
                      mlp2_w[i], mlp2_b[i].reshape(1, D)) for i in range(L)]

    for i in range(L):
        x = _mm_tc(hcur, lin1_w[i])
        parts = _gather_mul_scatter_sc(x, filts[i], row, col, zeros)
        hcur = _update_tc(parts[0], parts[1], hcur, lin2_w[i],
                          lin2_b[i].reshape(1, D), lin_w[i],
                          lin_b[i].reshape(1, D))

    return _head_tc(hcur, out1_w, out1_b.reshape(1, D // 2),
                    out2_w, out2_b.reshape(1, D))
